# Initial kernel scaffold; baseline (speedup 1.0000x reference)
import jax
import jax.numpy as jnp
from jax.experimental import pallas as pl


def kernel(x, edge_index, pseudo, weight, root_weight, bias):
    raise NotImplementedError("write your pallas kernel here")


# speedup vs baseline: 1.0408x; 1.0408x over previous
import functools

import jax
import jax.numpy as jnp
from jax import lax
from jax.experimental import pallas as pl
from jax.experimental.pallas import tpu as pltpu
from jax.experimental.pallas import tpu_sc as plsc

N = 10000
E = 320000
F = 128
K5 = 5
KA = 26
NS = 16
NC = 2
NT = NC * NS
EPT = 10240
EP = NT * EPT
CH = 32
SUP = 2048
CPS = SUP // CH
NSUP = EPT // SUP
FD = F + 16
NACC = 10240
ROWS_PT = NACC // NS
BN = 400
NB = N // BN


def _mm_body(x_ref, w_ref, o_ref):
    o_ref[0] = jnp.dot(x_ref[...], w_ref[0], preferred_element_type=jnp.float32)


@jax.jit
def _mm_call(x, wfull):
    return pl.pallas_call(
        _mm_body,
        grid=(NB, KA),
        in_specs=[
            pl.BlockSpec((BN, F), lambda nb, k: (nb, 0)),
            pl.BlockSpec((1, F, F), lambda nb, k: (k, 0, 0)),
        ],
        out_specs=pl.BlockSpec((1, BN, F), lambda nb, k: (k, nb, 0)),
        out_shape=jax.ShapeDtypeStruct((KA, N, F), jnp.float32),
    )(x, wfull)


def _combine_body(p_ref, r_ref, b_ref, o_ref):
    p = p_ref[...]
    s = p[0, :, :F] + p[1, :, :F]
    deg = p[0, :, F] + p[1, :, F]
    o_ref[...] = s / jnp.clip(deg, 1.0, None)[:, None] + r_ref[...] + b_ref[0]


@jax.jit
def _combine_call(partial, xt_flat, bias2):
    return pl.pallas_call(
        _combine_body,
        grid=(NB,),
        in_specs=[
            pl.BlockSpec((NC, BN, FD), lambda nb: (0, nb, 0)),
            pl.BlockSpec((BN, F), lambda nb: ((KA - 1) * N // BN + nb, 0)),
            pl.BlockSpec((1, F), lambda nb: (0, 0)),
        ],
        out_specs=pl.BlockSpec((BN, F), lambda nb: (nb, 0)),
        out_shape=jax.ShapeDtypeStruct((N, F), jnp.float32),
    )(partial, xt_flat, bias2)


def _sc_body(xt_ref, row2_ref, col_ref, px_ref, py_ref, z_ref, out_ref,
             row_st, col_st, px_st, py_st, gidx, basv, rows, oute, acc, sem):
    c = lax.axis_index("c")
    sid = lax.axis_index("s")
    tlin = c * NS + sid

    arow = pl.multiple_of(sid * ROWS_PT, ROWS_PT)
    pltpu.sync_copy(z_ref.at[pl.ds(arow, ROWS_PT)], acc.at[pl.ds(arow, ROWS_PT)])
    plsc.subcore_barrier()

    def sup_body(sp, carry):
        ebase = tlin * EPT + sp * SUP
        rbase = pl.multiple_of(ebase // CH, CPS)
        eoff = pl.multiple_of(ebase, SUP)
        pltpu.sync_copy(row2_ref.at[pl.ds(rbase, CPS)], row_st)
        pltpu.sync_copy(col_ref.at[pl.ds(eoff, SUP)], col_st)
        pltpu.sync_copy(px_ref.at[pl.ds(eoff, SUP)], px_st)
        pltpu.sync_copy(py_ref.at[pl.ds(eoff, SUP)], py_st)

        def ch_body(ci, carry2):
            for h in range(2):
                off = pl.multiple_of(ci * CH + h * 16, 16)
                colv = col_st[pl.ds(off, 16)]
                pxv = px_st[pl.ds(off, 16)]
                pyv = py_st[pl.ds(off, 16)]
                gid = ebase + ci * CH + h * 16 + lax.iota(jnp.int32, 16)
                msk = gid < E
                vx = pxv * float(K5 - 1)
                vy = pyv * float(K5 - 1)
                bxi = vx.astype(jnp.int32)
                byi = vy.astype(jnp.int32)
                fx = vx - bxi.astype(jnp.float32)
                fy = vy - byi.astype(jnp.float32)
                sumb = jnp.zeros((16,), jnp.float32)
                for s4 in range(4):
                    bit0, bit1 = s4 & 1, (s4 >> 1) & 1
                    wx = jnp.clip(bxi + bit0, 0, K5 - 1)
                    wy = jnp.clip(byi + bit1, 0, K5 - 1)
                    g = (wx + K5 * wy) * N + colv
                    bb = (fx if bit0 else 1.0 - fx) * (fy if bit1 else 1.0 - fy)
                    bb = jnp.where(msk, bb, 0.0)
                    gidx[0, pl.ds(s4 * CH + h * 16, 16)] = g
                    basv[0, pl.ds(s4 * CH + h * 16, 16)] = bb
                    sumb = sumb + bb

            pltpu.async_copy(xt_ref.at[gidx.at[0]], rows.at[0], sem).wait()

            for h in range(2):
                bv = [basv[0, pl.ds(s4 * CH + h * 16, 16)] for s4 in range(4)]
                lane0 = lax.iota(jnp.int32, 16) == 0
                for el in range(16):
                    e = h * 16 + el
                    b0, b1, b2, b3 = bv[0][el], bv[1][el], bv[2][el], bv[3][el]
                    for j in range(F // 16):
                        sl = pl.ds(j * 16, 16)
                        v = (b0 * rows[0, e, sl] + b1 * rows[0, CH + e, sl]
                             + b2 * rows[0, 2 * CH + e, sl]
                             + b3 * rows[0, 3 * CH + e, sl])
                        oute[0, e, sl] = v
                    oute[0, e, pl.ds(F, 16)] = jnp.where(
                        lane0, b0 + b1 + b2 + b3, 0.0)

            pltpu.sync_copy(oute.at[0], acc.at[row_st.at[ci]], add=True)
            return carry2
        lax.fori_loop(0, CPS, ch_body, 0)
        return carry
    lax.fori_loop(0, NSUP, sup_body, 0)

    plsc.subcore_barrier()
    pltpu.sync_copy(acc.at[pl.ds(arow, ROWS_PT)],
                    out_ref.at[c, pl.ds(arow, ROWS_PT)])


@jax.jit
def _sc_call(xt_flat, row2, colp, pxp, pyp, zeros):
    f = functools.partial(
        pl.kernel,
        out_type=jax.ShapeDtypeStruct((NC, NACC, FD), jnp.float32),
        mesh=plsc.VectorSubcoreMesh(core_axis_name="c", subcore_axis_name="s"),
        compiler_params=pltpu.CompilerParams(use_tc_tiling_on_sc=False),
        scratch_types=[
            pltpu.VMEM((CPS, CH), jnp.int32),
            pltpu.VMEM((SUP,), jnp.int32),
            pltpu.VMEM((SUP,), jnp.float32),
            pltpu.VMEM((SUP,), jnp.float32),
            pltpu.VMEM((1, 4 * CH), jnp.int32),
            pltpu.VMEM((1, 4 * CH), jnp.float32),
            pltpu.VMEM((1, 4 * CH, F), jnp.float32),
            pltpu.VMEM((1, CH, FD), jnp.float32),
            pltpu.VMEM_SHARED((NACC, FD), jnp.float32),
            pltpu.SemaphoreType.DMA,
        ],
    )(_sc_body)
    return f(xt_flat, row2, colp, pxp, pyp, zeros)


def kernel(x, edge_index, pseudo, weight, root_weight, bias):
    wfull = jnp.concatenate([weight, root_weight[None]], axis=0)
    xt = _mm_call(x, wfull)
    xt_flat = xt.reshape(KA * N, F)
    pad = jnp.zeros((EP - E,), jnp.int32)
    padf = jnp.zeros((EP - E,), jnp.float32)
    row2 = jnp.concatenate([edge_index[0], pad]).reshape(EP // CH, CH)
    colp = jnp.concatenate([edge_index[1], pad])
    pxp = jnp.concatenate([pseudo[:, 0], padf])
    pyp = jnp.concatenate([pseudo[:, 1], padf])
    zeros = jnp.zeros((NACC, FD), jnp.float32)
    partial = _sc_call(xt_flat, row2, colp, pxp, pyp, zeros)
    return _combine_call(partial, xt_flat, bias.reshape(1, F))


# speedup vs baseline: 1.3051x; 1.2540x over previous
import functools

import jax
import jax.numpy as jnp
from jax import lax
from jax.experimental import pallas as pl
from jax.experimental.pallas import tpu as pltpu
from jax.experimental.pallas import tpu_sc as plsc

N = 10000
E = 320000
F = 128
K5 = 5
KA = 26
NS = 16
NC = 2
NT = NC * NS
EPT = 10240
EP = NT * EPT
CH = 16
NCH = EPT // CH
SUP = 2048
CPS = SUP // CH
NSUP = EPT // SUP
FD = F + 16
ROWS_PT = N // NS
BN = 400
NB = N // BN


def _mm_body(x_ref, w_ref, o_ref):
    o_ref[0] = jnp.dot(x_ref[...], w_ref[0], preferred_element_type=jnp.float32)


@jax.jit
def _mm_call(x, wfull):
    return pl.pallas_call(
        _mm_body,
        grid=(NB, KA),
        in_specs=[
            pl.BlockSpec((BN, F), lambda nb, k: (nb, 0)),
            pl.BlockSpec((1, F, F), lambda nb, k: (k, 0, 0)),
        ],
        out_specs=pl.BlockSpec((1, BN, F), lambda nb, k: (k, nb, 0)),
        out_shape=jax.ShapeDtypeStruct((KA, N, F), jnp.float32),
    )(x, wfull)


def _combine_body(p_ref, r_ref, b_ref, o_ref):
    p = p_ref[...]
    s = p[0, :, :F] + p[1, :, :F]
    deg = p[0, :, F] + p[1, :, F]
    o_ref[...] = s / jnp.clip(deg, 1.0, None)[:, None] + r_ref[...] + b_ref[0]


@jax.jit
def _combine_call(partial, xt_flat, bias2):
    return pl.pallas_call(
        _combine_body,
        grid=(NB,),
        in_specs=[
            pl.BlockSpec((NC, BN, FD), lambda nb: (0, nb, 0)),
            pl.BlockSpec((BN, F), lambda nb: ((KA - 1) * N // BN + nb, 0)),
            pl.BlockSpec((1, F), lambda nb: (0, 0)),
        ],
        out_specs=pl.BlockSpec((BN, F), lambda nb: (nb, 0)),
        out_shape=jax.ShapeDtypeStruct((N, F), jnp.float32),
    )(partial, xt_flat, bias2)


def _sc_body(xt_ref, row2_ref, col_ref, px_ref, py_ref, z_ref, out_ref,
             row_st, col_st, px_st, py_st, gidx, basv, rows, oute, acc,
             sem_g, sem_s):
    c = lax.axis_index("c")
    sid = lax.axis_index("s")
    tlin = c * NS + sid
    tbase = pl.multiple_of(tlin * EPT, EPT)

    arow = pl.multiple_of(sid * ROWS_PT, ROWS_PT)
    pltpu.sync_copy(z_ref.at[pl.ds(arow, ROWS_PT)], acc.at[pl.ds(arow, ROWS_PT)])
    plsc.subcore_barrier()

    def stage(sp):
        rrow = pl.multiple_of(tlin * NCH, NCH) + sp * CPS
        pltpu.sync_copy(row2_ref.at[pl.ds(rrow, CPS)], row_st.at[sp % 2])
        eoff = pl.multiple_of(tbase + sp * SUP, SUP)
        pltpu.sync_copy(col_ref.at[pl.ds(eoff, SUP)], col_st)
        pltpu.sync_copy(px_ref.at[pl.ds(eoff, SUP)], px_st)
        pltpu.sync_copy(py_ref.at[pl.ds(eoff, SUP)], py_st)

    def compute_idx(cin, bi):
        off = pl.multiple_of((cin % CPS) * CH, CH)
        colv = col_st[pl.ds(off, 16)]
        pxv = px_st[pl.ds(off, 16)]
        pyv = py_st[pl.ds(off, 16)]
        gid = tbase + cin * CH + lax.iota(jnp.int32, 16)
        msk = gid < E
        vx = pxv * float(K5 - 1)
        vy = pyv * float(K5 - 1)
        bxi = vx.astype(jnp.int32)
        byi = vy.astype(jnp.int32)
        fx = vx - bxi.astype(jnp.float32)
        fy = vy - byi.astype(jnp.float32)
        for s4 in range(4):
            bit0, bit1 = s4 & 1, (s4 >> 1) & 1
            wx = jnp.clip(bxi + bit0, 0, K5 - 1)
            wy = jnp.clip(byi + bit1, 0, K5 - 1)
            g = (wx + K5 * wy) * N + colv
            bb = (fx if bit0 else 1.0 - fx) * (fy if bit1 else 1.0 - fy)
            bb = jnp.where(msk, bb, 0.0)
            gidx[bi, pl.ds(s4 * CH, 16)] = g
            basv[bi, pl.ds(s4 * CH, 16)] = bb

    def gather_desc(bi):
        return pltpu.make_async_copy(xt_ref.at[gidx.at[bi]], rows.at[bi],
                                     sem_g.at[bi])

    def scatter_wait(bi):
        pltpu.make_async_copy(oute.at[bi], acc.at[row_st.at[0, 0]],
                              sem_s.at[bi]).wait()

    stage(0)
    compute_idx(0, 0)
    gather_desc(0).start()

    def ch_body(ci, carry):
        b = ci & 1
        nb = 1 - b

        @pl.when(ci + 1 < NCH)
        def _():
            @pl.when((ci + 1) % CPS == 0)
            def _():
                stage((ci + 1) // CPS)
            compute_idx(ci + 1, nb)
            gather_desc(nb).start()

        gather_desc(b).wait()

        @pl.when(ci >= 2)
        def _():
            scatter_wait(b)

        bv = [basv[b, pl.ds(s4 * CH, 16)] for s4 in range(4)]
        lane0 = lax.iota(jnp.int32, 16) == 0
        for e in range(CH):
            b0, b1, b2, b3 = bv[0][e], bv[1][e], bv[2][e], bv[3][e]
            for j in range(F // 16):
                sl = pl.ds(j * 16, 16)
                v = (b0 * rows[b, e, sl] + b1 * rows[b, CH + e, sl]
                     + b2 * rows[b, 2 * CH + e, sl]
                     + b3 * rows[b, 3 * CH + e, sl])
                oute[b, e, sl] = v
            oute[b, e, pl.ds(F, 16)] = jnp.where(
                lane0, b0 + b1 + b2 + b3, 0.0)

        pltpu.async_copy(oute.at[b],
                         acc.at[row_st.at[(ci // CPS) % 2, ci % CPS]],
                         sem_s.at[b], add=True)
        return carry
    lax.fori_loop(0, NCH, ch_body, 0)

    scatter_wait(0)
    scatter_wait(1)

    plsc.subcore_barrier()
    pltpu.sync_copy(acc.at[pl.ds(arow, ROWS_PT)],
                    out_ref.at[c, pl.ds(arow, ROWS_PT)])


@jax.jit
def _sc_call(xt_flat, row2, colp, pxp, pyp, zeros):
    f = functools.partial(
        pl.kernel,
        out_type=jax.ShapeDtypeStruct((NC, N, FD), jnp.float32),
        mesh=plsc.VectorSubcoreMesh(core_axis_name="c", subcore_axis_name="s"),
        compiler_params=pltpu.CompilerParams(use_tc_tiling_on_sc=False),
        scratch_types=[
            pltpu.VMEM((2, CPS, CH), jnp.int32),
            pltpu.VMEM((SUP,), jnp.int32),
            pltpu.VMEM((SUP,), jnp.float32),
            pltpu.VMEM((SUP,), jnp.float32),
            pltpu.VMEM((2, 4 * CH), jnp.int32),
            pltpu.VMEM((2, 4 * CH), jnp.float32),
            pltpu.VMEM((2, 4 * CH, F), jnp.float32),
            pltpu.VMEM((2, CH, FD), jnp.float32),
            pltpu.VMEM_SHARED((N, FD), jnp.float32),
            pltpu.SemaphoreType.DMA((2,)),
            pltpu.SemaphoreType.DMA((2,)),
        ],
    )(_sc_body)
    return f(xt_flat, row2, colp, pxp, pyp, zeros)


def kernel(x, edge_index, pseudo, weight, root_weight, bias):
    wfull = jnp.concatenate([weight, root_weight[None]], axis=0)
    xt = _mm_call(x, wfull)
    xt_flat = xt.reshape(KA * N, F)
    pad = jnp.zeros((EP - E,), jnp.int32)
    padf = jnp.zeros((EP - E,), jnp.float32)
    row2 = jnp.concatenate([edge_index[0], pad]).reshape(EP // CH, CH)
    colp = jnp.concatenate([edge_index[1], pad])
    pxp = jnp.concatenate([pseudo[:, 0], padf])
    pyp = jnp.concatenate([pseudo[:, 1], padf])
    zeros = jnp.zeros((N, FD), jnp.float32)
    partial = _sc_call(xt_flat, row2, colp, pxp, pyp, zeros)
    return _combine_call(partial, xt_flat, bias.reshape(1, F))


# speedup vs baseline: 1.7351x; 1.3294x over previous
import functools

import jax
import jax.numpy as jnp
from jax import lax
from jax.experimental import pallas as pl
from jax.experimental.pallas import tpu as pltpu
from jax.experimental.pallas import tpu_sc as plsc

N = 10000
E = 320000
F = 128
K5 = 5
KA = 26
NS = 16
NC = 2
NT = NC * NS
EPT = 10240
EP = NT * EPT
CH = 16
NCH = EPT // CH
SUP = 2048
CPS = SUP // CH
NSUP = EPT // SUP
FD = F + 16
ROWS_PT = N // NS
BN = 400
NB = N // BN


def _mm_body(x_ref, w_ref, o_ref):
    o_ref[0] = jnp.dot(x_ref[...], w_ref[0], preferred_element_type=jnp.float32)


@jax.jit
def _mm_call(x, wfull):
    return pl.pallas_call(
        _mm_body,
        grid=(KA,),
        in_specs=[
            pl.BlockSpec((N, F), lambda k: (0, 0)),
            pl.BlockSpec((1, F, F), lambda k: (k, 0, 0)),
        ],
        out_specs=pl.BlockSpec((1, N, F), lambda k: (k, 0, 0)),
        out_shape=jax.ShapeDtypeStruct((KA, N, F), jnp.float32),
    )(x, wfull)


def _combine_body(p_ref, r_ref, b_ref, o_ref):
    p = p_ref[...]
    s = p[0, :, :F] + p[1, :, :F]
    deg = p[0, :, F] + p[1, :, F]
    o_ref[...] = s / jnp.clip(deg, 1.0, None)[:, None] + r_ref[...] + b_ref[0]


@jax.jit
def _combine_call(partial, xt_flat, bias2):
    return pl.pallas_call(
        _combine_body,
        grid=(NB,),
        in_specs=[
            pl.BlockSpec((NC, BN, FD), lambda nb: (0, nb, 0)),
            pl.BlockSpec((BN, F), lambda nb: ((KA - 1) * N // BN + nb, 0)),
            pl.BlockSpec((1, F), lambda nb: (0, 0)),
        ],
        out_specs=pl.BlockSpec((BN, F), lambda nb: (nb, 0)),
        out_shape=jax.ShapeDtypeStruct((N, F), jnp.float32),
    )(partial, xt_flat, bias2)


def _sc_body(xt_ref, row2_ref, col_ref, px_ref, py_ref, z_ref, out_ref,
             row_st, col_st, px_st, py_st, gidx, basv, rows, oute, acc,
             sem_g, sem_s):
    c = lax.axis_index("c")
    sid = lax.axis_index("s")
    tlin = c * NS + sid
    tbase = pl.multiple_of(tlin * EPT, EPT)

    arow = pl.multiple_of(sid * ROWS_PT, ROWS_PT)
    pltpu.sync_copy(z_ref.at[pl.ds(arow, ROWS_PT)], acc.at[pl.ds(arow, ROWS_PT)])
    plsc.subcore_barrier()

    def stage(sp):
        rrow = pl.multiple_of(tlin * NCH, NCH) + sp * CPS
        pltpu.sync_copy(row2_ref.at[pl.ds(rrow, CPS)], row_st.at[sp % 2])
        eoff = pl.multiple_of(tbase + sp * SUP, SUP)
        pltpu.sync_copy(col_ref.at[pl.ds(eoff, SUP)], col_st)
        pltpu.sync_copy(px_ref.at[pl.ds(eoff, SUP)], px_st)
        pltpu.sync_copy(py_ref.at[pl.ds(eoff, SUP)], py_st)

    def compute_idx(cin, bi):
        off = pl.multiple_of((cin % CPS) * CH, CH)
        colv = col_st[pl.ds(off, 16)]
        pxv = px_st[pl.ds(off, 16)]
        pyv = py_st[pl.ds(off, 16)]
        gid = tbase + cin * CH + lax.iota(jnp.int32, 16)
        msk = gid < E
        vx = pxv * float(K5 - 1)
        vy = pyv * float(K5 - 1)
        bxi = vx.astype(jnp.int32)
        byi = vy.astype(jnp.int32)
        fx = vx - bxi.astype(jnp.float32)
        fy = vy - byi.astype(jnp.float32)
        for s4 in range(4):
            bit0, bit1 = s4 & 1, (s4 >> 1) & 1
            wx = jnp.clip(bxi + bit0, 0, K5 - 1)
            wy = jnp.clip(byi + bit1, 0, K5 - 1)
            g = (wx + K5 * wy) * N + colv
            bb = (fx if bit0 else 1.0 - fx) * (fy if bit1 else 1.0 - fy)
            bb = jnp.where(msk, bb, 0.0)
            gidx[bi, pl.ds(s4 * CH, 16)] = g
            basv[bi, pl.ds(s4 * CH, 16)] = bb

    def gather_desc(bi):
        return pltpu.make_async_copy(xt_ref.at[gidx.at[bi]], rows.at[bi],
                                     sem_g.at[bi])

    def scatter_wait(bi):
        pltpu.make_async_copy(oute.at[bi], acc.at[row_st.at[0, 0]],
                              sem_s.at[bi]).wait()

    stage(0)
    compute_idx(0, 0)
    gather_desc(0).start()

    def ch_body(ci, carry):
        b = ci & 1
        nb = 1 - b

        @pl.when(ci + 1 < NCH)
        def _():
            @pl.when((ci + 1) % CPS == 0)
            def _():
                stage((ci + 1) // CPS)
            compute_idx(ci + 1, nb)
            gather_desc(nb).start()

        gather_desc(b).wait()

        @pl.when(ci >= 2)
        def _():
            scatter_wait(b)

        bv = [basv[b, pl.ds(s4 * CH, 16)] for s4 in range(4)]
        lane0 = lax.iota(jnp.int32, 16) == 0
        for e in range(CH):
            b0, b1, b2, b3 = bv[0][e], bv[1][e], bv[2][e], bv[3][e]
            for j in range(F // 16):
                sl = pl.ds(j * 16, 16)
                v = (b0 * rows[b, e, sl] + b1 * rows[b, CH + e, sl]
                     + b2 * rows[b, 2 * CH + e, sl]
                     + b3 * rows[b, 3 * CH + e, sl])
                oute[b, e, sl] = v
            oute[b, e, pl.ds(F, 16)] = jnp.where(
                lane0, b0 + b1 + b2 + b3, 0.0)

        pltpu.async_copy(oute.at[b],
                         acc.at[row_st.at[(ci // CPS) % 2, ci % CPS]],
                         sem_s.at[b], add=True)
        return carry
    lax.fori_loop(0, NCH, ch_body, 0)

    scatter_wait(0)
    scatter_wait(1)

    plsc.subcore_barrier()
    pltpu.sync_copy(acc.at[pl.ds(arow, ROWS_PT)],
                    out_ref.at[c, pl.ds(arow, ROWS_PT)])


@jax.jit
def _sc_call(xt_flat, row2, colp, pxp, pyp, zeros):
    f = functools.partial(
        pl.kernel,
        out_type=jax.ShapeDtypeStruct((NC, N, FD), jnp.float32),
        mesh=plsc.VectorSubcoreMesh(core_axis_name="c", subcore_axis_name="s"),
        compiler_params=pltpu.CompilerParams(use_tc_tiling_on_sc=False),
        scratch_types=[
            pltpu.VMEM((2, CPS, CH), jnp.int32),
            pltpu.VMEM((SUP,), jnp.int32),
            pltpu.VMEM((SUP,), jnp.float32),
            pltpu.VMEM((SUP,), jnp.float32),
            pltpu.VMEM((2, 4 * CH), jnp.int32),
            pltpu.VMEM((2, 4 * CH), jnp.float32),
            pltpu.VMEM((2, 4 * CH, F), jnp.float32),
            pltpu.VMEM((2, CH, FD), jnp.float32),
            pltpu.VMEM_SHARED((N, FD), jnp.float32),
            pltpu.SemaphoreType.DMA((2,)),
            pltpu.SemaphoreType.DMA((2,)),
        ],
    )(_sc_body)
    return f(xt_flat, row2, colp, pxp, pyp, zeros)


def kernel(x, edge_index, pseudo, weight, root_weight, bias):
    wfull = jnp.concatenate([weight, root_weight[None]], axis=0)
    xt = _mm_call(x, wfull)
    xt_flat = xt.reshape(KA * N, F)
    pad = jnp.zeros((EP - E,), jnp.int32)
    padf = jnp.zeros((EP - E,), jnp.float32)
    row2 = jnp.concatenate([edge_index[0], pad]).reshape(EP // CH, CH)
    colp = jnp.concatenate([edge_index[1], pad])
    pxp = jnp.concatenate([pseudo[:, 0], padf])
    pyp = jnp.concatenate([pseudo[:, 1], padf])
    zeros = jnp.zeros((N, FD), jnp.float32)
    partial = _sc_call(xt_flat, row2, colp, pxp, pyp, zeros)
    return _combine_call(partial, xt_flat, bias.reshape(1, F))


# speedup vs baseline: 2.1328x; 1.2292x over previous
import functools

import jax
import jax.numpy as jnp
from jax import lax
from jax.experimental import pallas as pl
from jax.experimental.pallas import tpu as pltpu
from jax.experimental.pallas import tpu_sc as plsc

N = 10000
E = 320000
F = 128
K5 = 5
KA = 26
NS = 16
NC = 2
NT = NC * NS
EPT = 10240
EP = NT * EPT
CH = 16
NCH = EPT // CH
SUP = 2048
CPS = SUP // CH
NSUP = EPT // SUP
FD = F + 16
ROWS_PT = N // NS
BN = 400
NB = N // BN


def _mm_body(x_ref, w_ref, o_ref):
    o_ref[0] = jnp.dot(x_ref[...], w_ref[0], preferred_element_type=jnp.float32)


@jax.jit
def _mm_call(x, wfull):
    return pl.pallas_call(
        _mm_body,
        grid=(KA,),
        in_specs=[
            pl.BlockSpec((N, F), lambda k: (0, 0)),
            pl.BlockSpec((1, F, F), lambda k: (k, 0, 0)),
        ],
        out_specs=pl.BlockSpec((1, N, F), lambda k: (k, 0, 0)),
        out_shape=jax.ShapeDtypeStruct((KA, N, F), jnp.float32),
    )(x, wfull)


def _combine_body(p_ref, r_ref, b_ref, o_ref):
    p = p_ref[...]
    s = p[0, :, :F] + p[1, :, :F]
    deg = p[0, :, F] + p[1, :, F]
    o_ref[...] = s / jnp.clip(deg, 1.0, None)[:, None] + r_ref[...] + b_ref[0]


@jax.jit
def _combine_call(partial, xt_flat, bias2):
    return pl.pallas_call(
        _combine_body,
        grid=(NB,),
        in_specs=[
            pl.BlockSpec((NC, BN, FD), lambda nb: (0, nb, 0)),
            pl.BlockSpec((BN, F), lambda nb: ((KA - 1) * N // BN + nb, 0)),
            pl.BlockSpec((1, F), lambda nb: (0, 0)),
        ],
        out_specs=pl.BlockSpec((BN, F), lambda nb: (nb, 0)),
        out_shape=jax.ShapeDtypeStruct((N, F), jnp.float32),
    )(partial, xt_flat, bias2)


def _sc_body(xt_ref, row2_ref, col_ref, px_ref, py_ref, z_ref, out_ref,
             row_st, col_st, px_st, py_st, gidx, basv, rows, oute, acc,
             sem_g, sem_s):
    c = lax.axis_index("c")
    sid = lax.axis_index("s")
    tlin = c * NS + sid
    tbase = pl.multiple_of(tlin * EPT, EPT)

    arow = pl.multiple_of(sid * ROWS_PT, ROWS_PT)
    pltpu.sync_copy(z_ref.at[pl.ds(arow, ROWS_PT)], acc.at[pl.ds(arow, ROWS_PT)])
    plsc.subcore_barrier()

    def stage(sp):
        rrow = pl.multiple_of(tlin * NCH, NCH) + sp * CPS
        pltpu.sync_copy(row2_ref.at[pl.ds(rrow, CPS)], row_st.at[sp % 2])
        eoff = pl.multiple_of(tbase + sp * SUP, SUP)
        pltpu.sync_copy(col_ref.at[pl.ds(eoff, SUP)], col_st)
        pltpu.sync_copy(px_ref.at[pl.ds(eoff, SUP)], px_st)
        pltpu.sync_copy(py_ref.at[pl.ds(eoff, SUP)], py_st)

    def compute_idx(cin, bi):
        off = pl.multiple_of((cin % CPS) * CH, CH)
        colv = col_st[pl.ds(off, 16)]
        pxv = px_st[pl.ds(off, 16)]
        pyv = py_st[pl.ds(off, 16)]
        gid = tbase + cin * CH + lax.iota(jnp.int32, 16)
        msk = gid < E
        vx = pxv * float(K5 - 1)
        vy = pyv * float(K5 - 1)
        bxi = vx.astype(jnp.int32)
        byi = vy.astype(jnp.int32)
        fx = vx - bxi.astype(jnp.float32)
        fy = vy - byi.astype(jnp.float32)
        for s4 in range(4):
            bit0, bit1 = s4 & 1, (s4 >> 1) & 1
            wx = jnp.clip(bxi + bit0, 0, K5 - 1)
            wy = jnp.clip(byi + bit1, 0, K5 - 1)
            g = (wx + K5 * wy) * N + colv
            bb = (fx if bit0 else 1.0 - fx) * (fy if bit1 else 1.0 - fy)
            bb = jnp.where(msk, bb, 0.0)
            gidx[bi, pl.ds(s4 * CH, 16)] = g
            basv[bi, pl.ds(s4 * CH, 16)] = bb

    def gather_desc(bi):
        return pltpu.make_async_copy(xt_ref.at[gidx.at[bi]], rows.at[bi],
                                     sem_g.at[bi])

    def scatter_wait(bi):
        pltpu.make_async_copy(oute.at[bi], acc.at[row_st.at[0, 0]],
                              sem_s.at[bi]).wait()

    stage(0)
    compute_idx(0, 0)
    gather_desc(0).start()

    def ch_body(ci, carry):
        b = ci & 1
        nb = 1 - b

        @pl.when(ci + 1 < NCH)
        def _():
            @pl.when((ci + 1) % CPS == 0)
            def _():
                stage((ci + 1) // CPS)
            compute_idx(ci + 1, nb)
            gather_desc(nb).start()

        gather_desc(b).wait()

        @pl.when(ci >= 2)
        def _():
            scatter_wait(b)

        bv = [basv[b, pl.ds(s4 * CH, 16)] for s4 in range(4)]
        lane0 = lax.iota(jnp.int32, 16) == 0
        for e in range(CH):
            b0, b1, b2, b3 = bv[0][e], bv[1][e], bv[2][e], bv[3][e]
            r = [[rows[b, s4 * CH + e, pl.ds(j * 16, 16)] for s4 in range(4)]
                 for j in range(F // 16)]
            for j in range(F // 16):
                v = (b0 * r[j][0] + b1 * r[j][1]
                     + b2 * r[j][2] + b3 * r[j][3])
                oute[b, e, pl.ds(j * 16, 16)] = v
            oute[b, e, pl.ds(F, 16)] = jnp.where(
                lane0, b0 + b1 + b2 + b3, 0.0)

        pltpu.async_copy(oute.at[b],
                         acc.at[row_st.at[(ci // CPS) % 2, ci % CPS]],
                         sem_s.at[b], add=True)
        return carry
    lax.fori_loop(0, NCH, ch_body, 0)

    scatter_wait(0)
    scatter_wait(1)

    plsc.subcore_barrier()
    pltpu.sync_copy(acc.at[pl.ds(arow, ROWS_PT)],
                    out_ref.at[c, pl.ds(arow, ROWS_PT)])


@jax.jit
def _sc_call(xt_flat, row2, colp, pxp, pyp, zeros):
    f = functools.partial(
        pl.kernel,
        out_type=jax.ShapeDtypeStruct((NC, N, FD), jnp.float32),
        mesh=plsc.VectorSubcoreMesh(core_axis_name="c", subcore_axis_name="s"),
        compiler_params=pltpu.CompilerParams(use_tc_tiling_on_sc=False),
        scratch_types=[
            pltpu.VMEM((2, CPS, CH), jnp.int32),
            pltpu.VMEM((SUP,), jnp.int32),
            pltpu.VMEM((SUP,), jnp.float32),
            pltpu.VMEM((SUP,), jnp.float32),
            pltpu.VMEM((2, 4 * CH), jnp.int32),
            pltpu.VMEM((2, 4 * CH), jnp.float32),
            pltpu.VMEM((2, 4 * CH, F), jnp.float32),
            pltpu.VMEM((2, CH, FD), jnp.float32),
            pltpu.VMEM_SHARED((N, FD), jnp.float32),
            pltpu.SemaphoreType.DMA((2,)),
            pltpu.SemaphoreType.DMA((2,)),
        ],
    )(_sc_body)
    return f(xt_flat, row2, colp, pxp, pyp, zeros)


def kernel(x, edge_index, pseudo, weight, root_weight, bias):
    wfull = jnp.concatenate([weight, root_weight[None]], axis=0)
    xt = _mm_call(x, wfull)
    xt_flat = xt.reshape(KA * N, F)
    pad = jnp.zeros((EP - E,), jnp.int32)
    padf = jnp.zeros((EP - E,), jnp.float32)
    row2 = jnp.concatenate([edge_index[0], pad]).reshape(EP // CH, CH)
    colp = jnp.concatenate([edge_index[1], pad])
    pxp = jnp.concatenate([pseudo[:, 0], padf])
    pyp = jnp.concatenate([pseudo[:, 1], padf])
    zeros = jnp.zeros((N, FD), jnp.float32)
    partial = _sc_call(xt_flat, row2, colp, pxp, pyp, zeros)
    return _combine_call(partial, xt_flat, bias.reshape(1, F))


# speedup vs baseline: 2.7511x; 1.2899x over previous
import functools

import jax
import jax.numpy as jnp
from jax import lax
from jax.experimental import pallas as pl
from jax.experimental.pallas import tpu as pltpu
from jax.experimental.pallas import tpu_sc as plsc

N = 10000
E = 320000
F = 128
K5 = 5
KT = 25
NS = 16
NC = 2
NT = NC * NS
EPT = 10240
EP = NT * EPT
CH = 16
NCH = EPT // CH
SUP = 2048
CPS = SUP // CH
NBUF = 4
FD = F + 16
ROWS_PT = N // NS
BN = 400
NB = N // BN


def _mm_body(x_ref, w_ref, o_ref):
    o_ref[0] = jnp.dot(x_ref[...], w_ref[0],
                       preferred_element_type=jnp.float32).astype(jnp.bfloat16)


@jax.jit
def _mm_call(x, weight):
    return pl.pallas_call(
        _mm_body,
        grid=(KT,),
        in_specs=[
            pl.BlockSpec((N, F), lambda k: (0, 0)),
            pl.BlockSpec((1, F, F), lambda k: (k, 0, 0)),
        ],
        out_specs=pl.BlockSpec((1, N, F), lambda k: (k, 0, 0)),
        out_shape=jax.ShapeDtypeStruct((KT, N, F), jnp.bfloat16),
    )(x, weight)


def _combine_body(p_ref, x_ref, rw_ref, b_ref, o_ref):
    p = p_ref[...]
    s = p[0, :, :F] + p[1, :, :F]
    deg = p[0, :, F] + p[1, :, F]
    root = jnp.dot(x_ref[...], rw_ref[...], preferred_element_type=jnp.float32)
    o_ref[...] = s / jnp.clip(deg, 1.0, None)[:, None] + root + b_ref[0]


@jax.jit
def _combine_call(partial, x, rw, bias2):
    return pl.pallas_call(
        _combine_body,
        grid=(NB,),
        in_specs=[
            pl.BlockSpec((NC, BN, FD), lambda nb: (0, nb, 0)),
            pl.BlockSpec((BN, F), lambda nb: (nb, 0)),
            pl.BlockSpec((F, F), lambda nb: (0, 0)),
            pl.BlockSpec((1, F), lambda nb: (0, 0)),
        ],
        out_specs=pl.BlockSpec((BN, F), lambda nb: (nb, 0)),
        out_shape=jax.ShapeDtypeStruct((N, F), jnp.float32),
    )(partial, x, rw, bias2)


def _sc_body(xt_ref, row2_ref, col_ref, px_ref, py_ref, z_ref, out_ref,
             row_st, col_st, px_st, py_st, gidx, basv, rows, oute, acc,
             sem_g, sem_s):
    c = lax.axis_index("c")
    sid = lax.axis_index("s")
    tlin = c * NS + sid
    tbase = pl.multiple_of(tlin * EPT, EPT)

    arow = pl.multiple_of(sid * ROWS_PT, ROWS_PT)
    pltpu.sync_copy(z_ref.at[pl.ds(arow, ROWS_PT)], acc.at[pl.ds(arow, ROWS_PT)])
    plsc.subcore_barrier()

    def stage(sp):
        rrow = pl.multiple_of(tlin * NCH, NCH) + sp * CPS
        pltpu.sync_copy(row2_ref.at[pl.ds(rrow, CPS)], row_st.at[sp % 2])
        eoff = pl.multiple_of(tbase + sp * SUP, SUP)
        pltpu.sync_copy(col_ref.at[pl.ds(eoff, SUP)], col_st)
        pltpu.sync_copy(px_ref.at[pl.ds(eoff, SUP)], px_st)
        pltpu.sync_copy(py_ref.at[pl.ds(eoff, SUP)], py_st)

    def compute_idx(cin, bi):
        off = pl.multiple_of((cin % CPS) * CH, CH)
        colv = col_st[pl.ds(off, 16)]
        pxv = px_st[pl.ds(off, 16)]
        pyv = py_st[pl.ds(off, 16)]
        gid = tbase + cin * CH + lax.iota(jnp.int32, 16)
        msk = gid < E
        vx = pxv * float(K5 - 1)
        vy = pyv * float(K5 - 1)
        bxi = vx.astype(jnp.int32)
        byi = vy.astype(jnp.int32)
        fx = vx - bxi.astype(jnp.float32)
        fy = vy - byi.astype(jnp.float32)
        for s4 in range(4):
            bit0, bit1 = s4 & 1, (s4 >> 1) & 1
            wx = jnp.clip(bxi + bit0, 0, K5 - 1)
            wy = jnp.clip(byi + bit1, 0, K5 - 1)
            g = (wx + K5 * wy) * N + colv
            bb = (fx if bit0 else 1.0 - fx) * (fy if bit1 else 1.0 - fy)
            bb = jnp.where(msk, bb, 0.0)
            gidx[bi, pl.ds(s4 * CH, 16)] = g
            basv[bi, pl.ds(s4 * CH, 16)] = bb

    def gather_desc(bi):
        return pltpu.make_async_copy(xt_ref.at[gidx.at[bi]], rows.at[bi],
                                     sem_g.at[bi])

    def scatter_wait(bi):
        pltpu.make_async_copy(oute.at[bi], acc.at[row_st.at[0, 0]],
                              sem_s.at[bi]).wait()

    stage(0)
    for i in range(NBUF - 1):
        compute_idx(i, i)
        gather_desc(i).start()

    def ch_body(ci, carry):
        b = ci & (NBUF - 1)
        pb = (ci + NBUF - 1) & (NBUF - 1)

        @pl.when(ci + NBUF - 1 < NCH)
        def _():
            @pl.when((ci + NBUF - 1) % CPS == 0)
            def _():
                stage((ci + NBUF - 1) // CPS)
            compute_idx(ci + NBUF - 1, pb)
            gather_desc(pb).start()

        gather_desc(b).wait()

        @pl.when(ci >= NBUF)
        def _():
            scatter_wait(b)

        bv = [basv[b, pl.ds(s4 * CH, 16)] for s4 in range(4)]
        bp = [plsc.bitcast(
                  plsc.pack(v, v, format=plsc.PackFormat.INTERLEAVED),
                  jnp.int32)
              for v in bv]
        lane0 = lax.iota(jnp.int32, 16) == 0
        zero16 = jnp.zeros((16,), jnp.int32)
        for e in range(CH):
            b0, b1, b2, b3 = bv[0][e], bv[1][e], bv[2][e], bv[3][e]
            mm = [plsc.bitcast(zero16 + bp[s4][e], jnp.bfloat16)
                  for s4 in range(4)]
            r = [[rows[b, s4 * CH + e, pl.ds(j * 32, 32)] for s4 in range(4)]
                 for j in range(F // 32)]
            for j in range(F // 32):
                v = (mm[0] * r[j][0] + mm[1] * r[j][1]
                     + mm[2] * r[j][2] + mm[3] * r[j][3])
                vl, vh = plsc.unpack(v, format=plsc.PackFormat.INTERLEAVED)
                oute[b, e, pl.ds(j * 32, 16)] = vl
                oute[b, e, pl.ds(j * 32 + 16, 16)] = vh
            oute[b, e, pl.ds(F, 16)] = jnp.where(
                lane0, b0 + b1 + b2 + b3, 0.0)

        pltpu.async_copy(oute.at[b],
                         acc.at[row_st.at[(ci // CPS) % 2, ci % CPS]],
                         sem_s.at[b], add=True)
        return carry
    lax.fori_loop(0, NCH, ch_body, 0)

    for i in range(NBUF):
        scatter_wait(i)

    plsc.subcore_barrier()
    pltpu.sync_copy(acc.at[pl.ds(arow, ROWS_PT)],
                    out_ref.at[c, pl.ds(arow, ROWS_PT)])


@jax.jit
def _sc_call(xt_flat, row2, colp, pxp, pyp, zeros):
    f = functools.partial(
        pl.kernel,
        out_type=jax.ShapeDtypeStruct((NC, N, FD), jnp.float32),
        mesh=plsc.VectorSubcoreMesh(core_axis_name="c", subcore_axis_name="s"),
        compiler_params=pltpu.CompilerParams(use_tc_tiling_on_sc=False,
                                             needs_layout_passes=False),
        scratch_types=[
            pltpu.VMEM((2, CPS, CH), jnp.int32),
            pltpu.VMEM((SUP,), jnp.int32),
            pltpu.VMEM((SUP,), jnp.float32),
            pltpu.VMEM((SUP,), jnp.float32),
            pltpu.VMEM((NBUF, 4 * CH), jnp.int32),
            pltpu.VMEM((NBUF, 4 * CH), jnp.float32),
            pltpu.VMEM((NBUF, 4 * CH, F), jnp.bfloat16),
            pltpu.VMEM((NBUF, CH, FD), jnp.float32),
            pltpu.VMEM_SHARED((N, FD), jnp.float32),
            pltpu.SemaphoreType.DMA((NBUF,)),
            pltpu.SemaphoreType.DMA((NBUF,)),
        ],
    )(_sc_body)
    return f(xt_flat, row2, colp, pxp, pyp, zeros)


def kernel(x, edge_index, pseudo, weight, root_weight, bias):
    r = jnp.arange(F)
    g, rr = r // 32, r % 32
    inv = g * 32 + jnp.where(rr % 2 == 0, rr // 2, 16 + (rr - 1) // 2)
    xt = _mm_call(x, weight[:, :, inv])
    xt_flat = xt.reshape(KT * N, F)
    pad = jnp.zeros((EP - E,), jnp.int32)
    padf = jnp.zeros((EP - E,), jnp.float32)
    row2 = jnp.concatenate([edge_index[0], pad]).reshape(EP // CH, CH)
    colp = jnp.concatenate([edge_index[1], pad])
    pxp = jnp.concatenate([pseudo[:, 0], padf])
    pyp = jnp.concatenate([pseudo[:, 1], padf])
    zeros = jnp.zeros((N, FD), jnp.float32)
    partial = _sc_call(xt_flat, row2, colp, pxp, pyp, zeros)
    return _combine_call(partial, x, root_weight, bias.reshape(1, F))


# speedup vs baseline: 3.6127x; 1.3132x over previous
import functools

import jax
import jax.numpy as jnp
from jax import lax
from jax.experimental import pallas as pl
from jax.experimental.pallas import tpu as pltpu
from jax.experimental.pallas import tpu_sc as plsc

N = 10000
E = 320000
F = 128
K5 = 5
KT = 25
NS = 16
NC = 2
NT = NC * NS
EPT = 10240
EP = NT * EPT
CH = 16
NCH = EPT // CH
SUP = 2048
CPS = SUP // CH
NBUF = 4
FD = F + 16
ROWS_PT = N // NS
BN = 400
NB = N // BN


def _mm_body(x_ref, w_ref, o_ref):
    def pack_half(rows):
        o = jnp.dot(rows, w_ref[0], preferred_element_type=jnp.float32)
        bits = lax.bitcast_convert_type(o, jnp.uint32)
        bits = (bits + 0x7FFF + ((bits >> 16) & 1)) >> 16
        return (bits[:, 64:] << 16) | bits[:, :64]
    a = pack_half(x_ref[: N // 2, :])
    b = pack_half(x_ref[N // 2:, :])
    o_ref[0] = jnp.concatenate([a, b], axis=1)


@jax.jit
def _mm_call(x, weight):
    return pl.pallas_call(
        _mm_body,
        grid=(KT,),
        in_specs=[
            pl.BlockSpec((N, F), lambda k: (0, 0)),
            pl.BlockSpec((1, F, F), lambda k: (k, 0, 0)),
        ],
        out_specs=pl.BlockSpec((1, N // 2, F), lambda k: (k, 0, 0)),
        out_shape=jax.ShapeDtypeStruct((KT, N // 2, F), jnp.uint32),
    )(x, weight)


def _combine_body(p_ref, x_ref, rw_ref, b_ref, o_ref):
    p = p_ref[...]
    s = p[0, :, :F] + p[1, :, :F]
    deg = p[0, :, F] + p[1, :, F]
    root = jnp.dot(x_ref[...], rw_ref[...], preferred_element_type=jnp.float32)
    o_ref[...] = s / jnp.clip(deg, 1.0, None)[:, None] + root + b_ref[0]


@jax.jit
def _combine_call(partial, x, rw, bias2):
    return pl.pallas_call(
        _combine_body,
        grid=(NB,),
        in_specs=[
            pl.BlockSpec((NC, BN, FD), lambda nb: (0, nb, 0)),
            pl.BlockSpec((BN, F), lambda nb: (nb, 0)),
            pl.BlockSpec((F, F), lambda nb: (0, 0)),
            pl.BlockSpec((1, F), lambda nb: (0, 0)),
        ],
        out_specs=pl.BlockSpec((BN, F), lambda nb: (nb, 0)),
        out_shape=jax.ShapeDtypeStruct((N, F), jnp.float32),
    )(partial, x, rw, bias2)


def _sc_body(xt_ref, row2_ref, col_ref, px_ref, py_ref, z_ref, out_ref,
             row_st, col_st, px_st, py_st, gidx, basv, rows, oute, acc,
             sem_g, sem_s):
    c = lax.axis_index("c")
    sid = lax.axis_index("s")
    tlin = c * NS + sid
    tbase = pl.multiple_of(tlin * EPT, EPT)

    arow = pl.multiple_of(sid * ROWS_PT, ROWS_PT)
    pltpu.sync_copy(z_ref.at[pl.ds(arow, ROWS_PT)], acc.at[pl.ds(arow, ROWS_PT)])
    zf = jnp.zeros((16,), jnp.float32)
    for bi in range(NBUF):
        for e in range(CH):
            oute[bi, e, pl.ds(F, 16)] = zf
    plsc.subcore_barrier()

    def stage(sp):
        rrow = pl.multiple_of(tlin * NCH, NCH) + sp * CPS
        pltpu.sync_copy(row2_ref.at[pl.ds(rrow, CPS)], row_st.at[sp % 2])
        eoff = pl.multiple_of(tbase + sp * SUP, SUP)
        pltpu.sync_copy(col_ref.at[pl.ds(eoff, SUP)], col_st)
        pltpu.sync_copy(px_ref.at[pl.ds(eoff, SUP)], px_st)
        pltpu.sync_copy(py_ref.at[pl.ds(eoff, SUP)], py_st)

    def compute_idx(cin, bi):
        off = pl.multiple_of((cin % CPS) * CH, CH)
        colv = col_st[pl.ds(off, 16)]
        pxv = px_st[pl.ds(off, 16)]
        pyv = py_st[pl.ds(off, 16)]
        gid = tbase + cin * CH + lax.iota(jnp.int32, 16)
        msk = gid < E
        vx = pxv * float(K5 - 1)
        vy = pyv * float(K5 - 1)
        bxi = vx.astype(jnp.int32)
        byi = vy.astype(jnp.int32)
        fx = vx - bxi.astype(jnp.float32)
        fy = vy - byi.astype(jnp.float32)
        half = jnp.where(colv >= N // 2, 1, 0)
        cq = 2 * colv - (N - 1) * half
        for s4 in range(4):
            bit0, bit1 = s4 & 1, (s4 >> 1) & 1
            wx = jnp.clip(bxi + bit0, 0, K5 - 1)
            wy = jnp.clip(byi + bit1, 0, K5 - 1)
            g = (wx + K5 * wy) * N + cq
            bb = (fx if bit0 else 1.0 - fx) * (fy if bit1 else 1.0 - fy)
            bb = jnp.where(msk, bb, 0.0)
            gidx[bi, pl.ds(s4 * CH, 16)] = g
            basv[bi, pl.ds(s4 * CH, 16)] = bb

    def gather_desc(bi):
        return pltpu.make_async_copy(xt_ref.at[gidx.at[bi]], rows.at[bi],
                                     sem_g.at[bi])

    def scatter_wait(bi):
        pltpu.make_async_copy(oute.at[bi], acc.at[row_st.at[0, 0]],
                              sem_s.at[bi]).wait()

    stage(0)
    for i in range(NBUF - 1):
        compute_idx(i, i)
        gather_desc(i).start()

    def ch_body(ci, carry):
        b = ci & (NBUF - 1)
        pb = (ci + NBUF - 1) & (NBUF - 1)

        @pl.when(ci + NBUF - 1 < NCH)
        def _():
            @pl.when((ci + NBUF - 1) % CPS == 0)
            def _():
                stage((ci + NBUF - 1) // CPS)
            compute_idx(ci + NBUF - 1, pb)
            gather_desc(pb).start()

        gather_desc(b).wait()

        @pl.when(ci >= NBUF)
        def _():
            scatter_wait(b)

        bv = [basv[b, pl.ds(s4 * CH, 16)] for s4 in range(4)]
        bp = [plsc.bitcast(
                  plsc.pack(v, v, format=plsc.PackFormat.INTERLEAVED),
                  jnp.int32)
              for v in bv]
        zero16 = jnp.zeros((16,), jnp.int32)
        for e in range(CH):
            mm = [plsc.bitcast(zero16 + bp[s4][e], jnp.bfloat16)
                  for s4 in range(4)]
            r = [[plsc.bitcast(rows[b, s4 * CH + e, pl.ds(j * 16, 16)],
                               jnp.bfloat16) for s4 in range(4)]
                 for j in range(F // 32)]
            for j in range(F // 32):
                v = (mm[0] * r[j][0] + mm[1] * r[j][1]
                     + mm[2] * r[j][2] + mm[3] * r[j][3])
                vl, vh = plsc.unpack(v, format=plsc.PackFormat.INTERLEAVED)
                oute[b, e, pl.ds(j * 16, 16)] = vl
                oute[b, e, pl.ds(64 + j * 16, 16)] = vh
        sumb = bv[0] + bv[1] + bv[2] + bv[3]
        plsc.store_scatter(oute.at[b],
                           [lax.iota(jnp.int32, 16), zero16 + F], sumb)

        pltpu.async_copy(oute.at[b],
                         acc.at[row_st.at[(ci // CPS) % 2, ci % CPS]],
                         sem_s.at[b], add=True)
        return carry
    lax.fori_loop(0, NCH, ch_body, 0)

    for i in range(NBUF):
        scatter_wait(i)

    plsc.subcore_barrier()
    pltpu.sync_copy(acc.at[pl.ds(arow, ROWS_PT)],
                    out_ref.at[c, pl.ds(arow, ROWS_PT)])


@jax.jit
def _sc_call(xt_flat, row2, colp, pxp, pyp, zeros):
    f = functools.partial(
        pl.kernel,
        out_type=jax.ShapeDtypeStruct((NC, N, FD), jnp.float32),
        mesh=plsc.VectorSubcoreMesh(core_axis_name="c", subcore_axis_name="s"),
        compiler_params=pltpu.CompilerParams(use_tc_tiling_on_sc=False,
                                             needs_layout_passes=False),
        scratch_types=[
            pltpu.VMEM((2, CPS, CH), jnp.int32),
            pltpu.VMEM((SUP,), jnp.int32),
            pltpu.VMEM((SUP,), jnp.float32),
            pltpu.VMEM((SUP,), jnp.float32),
            pltpu.VMEM((NBUF, 4 * CH), jnp.int32),
            pltpu.VMEM((NBUF, 4 * CH), jnp.float32),
            pltpu.VMEM((NBUF, 4 * CH, F // 2), jnp.uint32),
            pltpu.VMEM((NBUF, CH, FD), jnp.float32),
            pltpu.VMEM_SHARED((N, FD), jnp.float32),
            pltpu.SemaphoreType.DMA((NBUF,)),
            pltpu.SemaphoreType.DMA((NBUF,)),
        ],
    )(_sc_body)
    return f(xt_flat, row2, colp, pxp, pyp, zeros)


def kernel(x, edge_index, pseudo, weight, root_weight, bias):
    xt = _mm_call(x, weight)
    xt_flat = xt.reshape(KT * N, F // 2)
    pad = jnp.zeros((EP - E,), jnp.int32)
    padf = jnp.zeros((EP - E,), jnp.float32)
    row2 = jnp.concatenate([edge_index[0], pad]).reshape(EP // CH, CH)
    colp = jnp.concatenate([edge_index[1], pad])
    pxp = jnp.concatenate([pseudo[:, 0], padf])
    pyp = jnp.concatenate([pseudo[:, 1], padf])
    zeros = jnp.zeros((N, FD), jnp.float32)
    partial = _sc_call(xt_flat, row2, colp, pxp, pyp, zeros)
    return _combine_call(partial, x, root_weight, bias.reshape(1, F))
